# Initial kernel scaffold; baseline (speedup 1.0000x reference)
import jax
import jax.numpy as jnp
from jax.experimental import pallas as pl


def kernel(node_scalars, node_vectors, edge_vec, scalar_edge_feats, lengths, W0_up, b0_up, W1_up, Wm0, bm0, Wm1, bm1, Wm2, bm2, W0_dn, b0_dn, W1_dn, senders, receivers):
    raise NotImplementedError("write your pallas kernel here")


# speedup vs baseline: 14.3451x; 14.3451x over previous
import functools

import jax
import jax.numpy as jnp
import numpy as np
from jax import lax
from jax.experimental import pallas as pl
from jax.experimental.pallas import tpu as pltpu
from jax.experimental.pallas import tpu_sc as plsc

N = 50000
E = 800000
C = 16
SE = 8
HID = 128
AVG = 16.0
INV = 1.0 / np.sqrt(C)
ISQ3 = 1.0 / np.sqrt(3.0)

NC = 2
NS = 16
NW = NC * NS

K = 128
NROWS = 6272
EPAD = NROWS * K
ROWS_W = NROWS // NW
ROWS_CORE = NROWS // NS
STRIPE = N // NS

BN = 2000
BEDGE = 2048

@functools.cache
def _sc_mesh():
    return plsc.VectorSubcoreMesh(
        core_axis_name="c", subcore_axis_name="s",
        num_cores=NC, num_subcores=NS)


def _linear_up_body(ns_ref, nvx_ref, nvy_ref, nvz_ref, w0_ref, b0_ref, w1_ref,
                    sv_ref, s_ref):
    s = jnp.dot(ns_ref[...], w0_ref[...],
                preferred_element_type=jnp.float32) * INV + b0_ref[...]
    vx = jnp.dot(nvx_ref[...], w1_ref[...],
                 preferred_element_type=jnp.float32) * INV
    vy = jnp.dot(nvy_ref[...], w1_ref[...],
                 preferred_element_type=jnp.float32) * INV
    vz = jnp.dot(nvz_ref[...], w1_ref[...],
                 preferred_element_type=jnp.float32) * INV
    sv_ref[...] = jnp.concatenate([s, vx, vy, vz], axis=1)
    s_ref[...] = s


def _linear_up(ns, nvx, nvy, nvz, w0, b0, w1):
    grid = (N // BN,)
    return pl.pallas_call(
        _linear_up_body,
        grid=grid,
        in_specs=[
            pl.BlockSpec((BN, C), lambda i: (i, 0)),
            pl.BlockSpec((BN, C), lambda i: (i, 0)),
            pl.BlockSpec((BN, C), lambda i: (i, 0)),
            pl.BlockSpec((BN, C), lambda i: (i, 0)),
            pl.BlockSpec((C, C), lambda i: (0, 0)),
            pl.BlockSpec((1, C), lambda i: (0, 0)),
            pl.BlockSpec((C, C), lambda i: (0, 0)),
        ],
        out_specs=[
            pl.BlockSpec((BN, 4 * C), lambda i: (i, 0)),
            pl.BlockSpec((BN, C), lambda i: (i, 0)),
        ],
        out_shape=[
            jax.ShapeDtypeStruct((N, 4 * C), jnp.float32),
            jax.ShapeDtypeStruct((N, C), jnp.float32),
        ],
    )(ns, nvx, nvy, nvz, w0, b0, w1)


def _gather_body(sv_hbm, s_hbm, snd_hbm, rcv_hbm, sndf_hbm, rcvf_hbm,
                 idx_v, rows64_v, rows16_v, sem):
    c = lax.axis_index("c")
    s = lax.axis_index("s")
    wid = s * NC + c
    base = wid * ROWS_W

    def body(j, carry):
        row = base + j
        pltpu.sync_copy(snd_hbm.at[row], idx_v)
        pltpu.async_copy(sv_hbm.at[idx_v], rows64_v, sem).wait()
        pltpu.sync_copy(rows64_v, sndf_hbm.at[pl.ds(row * K, K)])
        pltpu.sync_copy(rcv_hbm.at[row], idx_v)
        pltpu.async_copy(s_hbm.at[idx_v], rows16_v, sem).wait()
        pltpu.sync_copy(rows16_v, rcvf_hbm.at[pl.ds(row * K, K)])
        return carry

    lax.fori_loop(0, ROWS_W, body, 0)


@functools.cache
def _gather():
    return pl.kernel(
        _gather_body,
        out_type=[
            jax.ShapeDtypeStruct((EPAD, 4 * C), jnp.float32),
            jax.ShapeDtypeStruct((EPAD, C), jnp.float32),
        ],
        mesh=_sc_mesh(),
        scratch_types=[
            pltpu.VMEM((K,), jnp.int32),
            pltpu.VMEM((K, 4 * C), jnp.float32),
            pltpu.VMEM((K, C), jnp.float32),
            pltpu.SemaphoreType.DMA,
        ],
        compiler_params=pltpu.CompilerParams(use_tc_tiling_on_sc=False),
    )


def _edge_body(snd_ref, rcv_ref, ev_ref, sef_ref, len_ref,
               wm0_ref, bm0_ref, wm1_ref, bm1_ref, wm2_ref, bm2_ref, m_ref):
    snd = snd_ref[...]
    s_snd = snd[:, :C]
    vx = snd[:, C:2 * C]
    vy = snd[:, 2 * C:3 * C]
    vz = snd[:, 3 * C:]
    ev = ev_ref[...]
    evx = ev[:, 0:1]
    evy = ev[:, 1:2]
    evz = ev[:, 2:3]

    pad = jnp.zeros((BEDGE, 7), jnp.float32)
    mlp_in = jnp.concatenate(
        [s_snd, rcv_ref[...], sef_ref[...], len_ref[...], pad], axis=1)

    h = jnp.dot(mlp_in, wm0_ref[...],
                preferred_element_type=jnp.float32) + bm0_ref[...]
    h = h * (1.0 / (1.0 + jnp.exp(-h)))
    h = jnp.dot(h, wm1_ref[...],
                preferred_element_type=jnp.float32) + bm1_ref[...]
    h = h * (1.0 / (1.0 + jnp.exp(-h)))
    mix = jnp.dot(h, wm2_ref[...],
                  preferred_element_type=jnp.float32) + bm2_ref[...]

    mix_lo = mix[:, :C]
    mix_hi = mix[:, C:]
    m0 = (vx * evx + vy * evy + vz * evz) * (mix_lo * ISQ3)
    sm = s_snd * mix_hi
    m_ref[0] = jnp.concatenate([m0, sm * evx], axis=1)
    m_ref[1] = jnp.concatenate([sm * evy, sm * evz], axis=1)


def _edge_mlp(sndf, rcvf, ev, sef, ln, wm0p, bm0, wm1, bm1, wm2, bm2):
    grid = (EPAD // BEDGE,)
    wspec = lambda shape: pl.BlockSpec(shape, lambda i: (0,) * len(shape))
    return pl.pallas_call(
        _edge_body,
        grid=grid,
        in_specs=[
            pl.BlockSpec((BEDGE, 4 * C), lambda i: (i, 0)),
            pl.BlockSpec((BEDGE, C), lambda i: (i, 0)),
            pl.BlockSpec((BEDGE, 3), lambda i: (i, 0)),
            pl.BlockSpec((BEDGE, SE), lambda i: (i, 0)),
            pl.BlockSpec((BEDGE, 1), lambda i: (i, 0)),
            wspec((3 * C, HID)),
            wspec((1, HID)),
            wspec((HID, HID)),
            wspec((1, HID)),
            wspec((HID, 2 * C)),
            wspec((1, 2 * C)),
        ],
        out_specs=[pl.BlockSpec((2, BEDGE, 2 * C), lambda i: (0, i, 0))],
        out_shape=[jax.ShapeDtypeStruct((2, EPAD, 2 * C), jnp.float32)],
    )(sndf, rcvf, ev, sef, ln, wm0p, bm0, wm1, bm1, wm2, bm2)[0]


def _scatter_body(m_hbm, rcv_hbm, z_hbm, agg_hbm, acc_sh, idx_v, data_v):
    c = lax.axis_index("c")
    s = lax.axis_index("s")
    stripe = s * STRIPE
    pltpu.sync_copy(z_hbm, acc_sh.at[pl.ds(stripe, STRIPE)])
    plsc.subcore_barrier()

    def body(j, carry):
        grow = s * ROWS_CORE + j
        pltpu.sync_copy(rcv_hbm.at[grow], idx_v)
        pltpu.sync_copy(m_hbm.at[c, pl.ds(grow * K, K)], data_v)
        pltpu.sync_copy(data_v, acc_sh.at[idx_v], add=True)
        return carry

    lax.fori_loop(0, ROWS_CORE, body, 0)
    plsc.subcore_barrier()
    pltpu.sync_copy(acc_sh.at[pl.ds(stripe, STRIPE)],
                    agg_hbm.at[c, pl.ds(stripe, STRIPE)])


@functools.cache
def _scatter():
    return pl.kernel(
        _scatter_body,
        out_type=jax.ShapeDtypeStruct((2, N, 2 * C), jnp.float32),
        mesh=_sc_mesh(),
        scratch_types=[
            pltpu.VMEM_SHARED((N, 2 * C), jnp.float32),
            pltpu.VMEM((K,), jnp.int32),
            pltpu.VMEM((K, 2 * C), jnp.float32),
        ],
        compiler_params=pltpu.CompilerParams(use_tc_tiling_on_sc=False),
    )


def _linear_dn_body(agg_ref, w0_ref, b0_ref, w1_ref, out_ref):
    a = agg_ref[...]
    a0 = a[0, :, :C]
    ax = a[0, :, C:]
    ay = a[1, :, :C]
    az = a[1, :, C:]
    scale = INV / AVG
    o0 = jnp.dot(a0, w0_ref[...],
                 preferred_element_type=jnp.float32) * scale + b0_ref[...]
    ox = jnp.dot(ax, w1_ref[...], preferred_element_type=jnp.float32) * scale
    oy = jnp.dot(ay, w1_ref[...], preferred_element_type=jnp.float32) * scale
    oz = jnp.dot(az, w1_ref[...], preferred_element_type=jnp.float32) * scale
    out_ref[...] = jnp.concatenate([o0, ox, oy, oz], axis=1)


def _linear_dn(agg, w0, b0, w1):
    grid = (N // BN,)
    return pl.pallas_call(
        _linear_dn_body,
        grid=grid,
        in_specs=[
            pl.BlockSpec((2, BN, 2 * C), lambda i: (0, i, 0)),
            pl.BlockSpec((C, C), lambda i: (0, 0)),
            pl.BlockSpec((1, C), lambda i: (0, 0)),
            pl.BlockSpec((C, C), lambda i: (0, 0)),
        ],
        out_specs=[pl.BlockSpec((BN, 4 * C), lambda i: (i, 0))],
        out_shape=[jax.ShapeDtypeStruct((N, 4 * C), jnp.float32)],
    )(agg, w0, b0, w1)[0]


def kernel(node_scalars, node_vectors, edge_vec, scalar_edge_feats, lengths,
           W0_up, b0_up, W1_up, Wm0, bm0, Wm1, bm1, Wm2, bm2,
           W0_dn, b0_dn, W1_dn, senders, receivers):
    nvx = node_vectors[:, :, 0]
    nvy = node_vectors[:, :, 1]
    nvz = node_vectors[:, :, 2]
    sv, s16 = _linear_up(node_scalars, nvx, nvy, nvz,
                         W0_up, b0_up.reshape(1, C), W1_up)

    padn = EPAD - E
    snd2d = jnp.pad(senders, (0, padn)).reshape(NROWS, K)
    rcv2d = jnp.pad(receivers, (0, padn)).reshape(NROWS, K)
    evp = jnp.pad(edge_vec, ((0, padn), (0, 0)))
    sefp = jnp.pad(scalar_edge_feats, ((0, padn), (0, 0)))
    lnp = jnp.pad(lengths, ((0, padn), (0, 0)))

    sndf, rcvf = _gather()(sv, s16, snd2d, rcv2d)

    wm0p = jnp.concatenate([Wm0, jnp.zeros((3 * C - (2 * C + SE + 1), HID),
                                           jnp.float32)], axis=0)
    m2 = _edge_mlp(sndf, rcvf, evp, sefp, lnp,
                   wm0p, bm0.reshape(1, HID), Wm1, bm1.reshape(1, HID),
                   Wm2, bm2.reshape(1, 2 * C))

    agg = _scatter()(m2, rcv2d, jnp.zeros((STRIPE, 2 * C), jnp.float32))

    outax = _linear_dn(agg, W0_dn, b0_dn.reshape(1, C), W1_dn)
    o1 = outax[:, C:].reshape(N, 3, C).transpose(0, 2, 1).reshape(N, 3 * C)
    return jnp.concatenate([outax[:, :C], o1], axis=1)


# speedup vs baseline: 15.3352x; 1.0690x over previous
import functools

import jax
import jax.numpy as jnp
import numpy as np
from jax import lax
from jax.experimental import pallas as pl
from jax.experimental.pallas import tpu as pltpu
from jax.experimental.pallas import tpu_sc as plsc

N = 50000
E = 800000
C = 16
SE = 8
HID = 128
AVG = 16.0
INV = 1.0 / np.sqrt(C)
ISQ3 = 1.0 / np.sqrt(3.0)

NC = 2
NS = 16
NW = NC * NS

K = 128
NROWS = 6272
EPAD = NROWS * K
ROWS_W = NROWS // NW
ROWS_CORE = NROWS // NS
STRIPE = N // NS

BN = 2000
GB = 1024
NG = EPAD // (2 * GB)


@functools.cache
def _sc_mesh():
    return plsc.VectorSubcoreMesh(
        core_axis_name="c", subcore_axis_name="s",
        num_cores=NC, num_subcores=NS)


def _linear_up_body(ns_ref, nvx_ref, nvy_ref, nvz_ref, w0_ref, b0_ref, w1_ref,
                    sv_ref, s_ref):
    s = jnp.dot(ns_ref[...], w0_ref[...],
                preferred_element_type=jnp.float32) * INV + b0_ref[...]
    vx = jnp.dot(nvx_ref[...], w1_ref[...],
                 preferred_element_type=jnp.float32) * INV
    vy = jnp.dot(nvy_ref[...], w1_ref[...],
                 preferred_element_type=jnp.float32) * INV
    vz = jnp.dot(nvz_ref[...], w1_ref[...],
                 preferred_element_type=jnp.float32) * INV
    sv_ref[...] = jnp.concatenate([s, vx, vy, vz], axis=1)
    s_ref[...] = s


def _linear_up(ns, nvx, nvy, nvz, w0, b0, w1):
    grid = (N // BN,)
    return pl.pallas_call(
        _linear_up_body,
        grid=grid,
        in_specs=[
            pl.BlockSpec((BN, C), lambda i: (i, 0)),
            pl.BlockSpec((BN, C), lambda i: (i, 0)),
            pl.BlockSpec((BN, C), lambda i: (i, 0)),
            pl.BlockSpec((BN, C), lambda i: (i, 0)),
            pl.BlockSpec((C, C), lambda i: (0, 0)),
            pl.BlockSpec((1, C), lambda i: (0, 0)),
            pl.BlockSpec((C, C), lambda i: (0, 0)),
        ],
        out_specs=[
            pl.BlockSpec((BN, 4 * C), lambda i: (i, 0)),
            pl.BlockSpec((BN, C), lambda i: (i, 0)),
        ],
        out_shape=[
            jax.ShapeDtypeStruct((N, 4 * C), jnp.float32),
            jax.ShapeDtypeStruct((N, C), jnp.float32),
        ],
    )(ns, nvx, nvy, nvz, w0, b0, w1)


def _gather_body(sv_hbm, s_hbm, snd_hbm, rcv_hbm, sndf_hbm, rcvf_hbm,
                 idx_v, rows64_v, rows16_v, sem):
    c = lax.axis_index("c")
    s = lax.axis_index("s")
    wid = s * NC + c
    base = wid * ROWS_W

    def body(j, carry):
        row = base + j
        pltpu.sync_copy(snd_hbm.at[row], idx_v)
        pltpu.async_copy(sv_hbm.at[idx_v], rows64_v, sem).wait()
        pltpu.sync_copy(rows64_v, sndf_hbm.at[pl.ds(row * K, K)])
        pltpu.sync_copy(rcv_hbm.at[row], idx_v)
        pltpu.async_copy(s_hbm.at[idx_v], rows16_v, sem).wait()
        pltpu.sync_copy(rows16_v, rcvf_hbm.at[pl.ds(row * K, K)])
        return carry

    lax.fori_loop(0, ROWS_W, body, 0)


@functools.cache
def _gather():
    return pl.kernel(
        _gather_body,
        out_type=[
            jax.ShapeDtypeStruct((EPAD, 4 * C), jnp.float32),
            jax.ShapeDtypeStruct((EPAD, C), jnp.float32),
        ],
        mesh=_sc_mesh(),
        scratch_types=[
            pltpu.VMEM((K,), jnp.int32),
            pltpu.VMEM((K, 4 * C), jnp.float32),
            pltpu.VMEM((K, C), jnp.float32),
            pltpu.SemaphoreType.DMA,
        ],
        compiler_params=pltpu.CompilerParams(use_tc_tiling_on_sc=False),
    )


def _edge_group(snd, rcv, ev, sef, ln,
                wm0_ref, bm0_ref, wm1_ref, bm1_ref, wm2_ref, bm2_ref):
    s_snd = snd[:, :C]
    vx = snd[:, C:2 * C]
    vy = snd[:, 2 * C:3 * C]
    vz = snd[:, 3 * C:4 * C]
    evx = ev[:, 0:1]
    evy = ev[:, 1:2]
    evz = ev[:, 2:3]

    pad = jnp.zeros((GB, 7), jnp.float32)
    mlp_in = jnp.concatenate([s_snd, rcv, sef, ln, pad], axis=1)

    h = jnp.dot(mlp_in, wm0_ref[...],
                preferred_element_type=jnp.float32) + bm0_ref[...]
    h = h * (1.0 / (1.0 + jnp.exp(-h)))
    h = jnp.dot(h, wm1_ref[...],
                preferred_element_type=jnp.float32) + bm1_ref[...]
    h = h * (1.0 / (1.0 + jnp.exp(-h)))
    mix = jnp.dot(h, wm2_ref[...],
                  preferred_element_type=jnp.float32) + bm2_ref[...]

    mix_lo = mix[:, :C]
    mix_hi = mix[:, C:]
    m0 = (vx * evx + vy * evy + vz * evz) * (mix_lo * ISQ3)
    sm = s_snd * mix_hi
    return (jnp.concatenate([m0, sm * evx], axis=1),
            jnp.concatenate([sm * evy, sm * evz], axis=1))


def _edge_body(snd_ref, rcv0_ref, rcv1_ref, ev0_ref, ev1_ref,
               sef0_ref, sef1_ref, ln0_ref, ln1_ref,
               wm0_ref, bm0_ref, wm1_ref, bm1_ref, wm2_ref, bm2_ref, m_ref):
    snd = snd_ref[...]
    wrefs = (wm0_ref, bm0_ref, wm1_ref, bm1_ref, wm2_ref, bm2_ref)
    h0e, h1e = _edge_group(snd[:, :4 * C], rcv0_ref[...], ev0_ref[...],
                           sef0_ref[...], ln0_ref[...], *wrefs)
    h0o, h1o = _edge_group(snd[:, 4 * C:], rcv1_ref[...], ev1_ref[...],
                           sef1_ref[...], ln1_ref[...], *wrefs)
    hb = GB // 2
    m_ref[0] = jnp.concatenate(
        [h0e[:hb], h0o[:hb], h0e[hb:], h0o[hb:]], axis=1)
    m_ref[1] = jnp.concatenate(
        [h1e[:hb], h1o[:hb], h1e[hb:], h1o[hb:]], axis=1)


def _edge_mlp(sndp, rcvf, ev, sef, ln, wm0p, bm0, wm1, bm1, wm2, bm2):
    grid = (NG,)
    wspec = lambda shape: pl.BlockSpec(shape, lambda i: (0,) * len(shape))

    def gspec(width, g):
        return pl.BlockSpec((GB, width), lambda i, g=g: (2 * i + g, 0))

    in_specs = (
        [pl.BlockSpec((GB, 2 * 4 * C), lambda i: (i, 0))]
        + [gspec(C, g) for g in range(2)]
        + [gspec(3, g) for g in range(2)]
        + [gspec(SE, g) for g in range(2)]
        + [gspec(1, g) for g in range(2)]
        + [wspec((3 * C, HID)), wspec((1, HID)), wspec((HID, HID)),
           wspec((1, HID)), wspec((HID, 2 * C)), wspec((1, 2 * C))]
    )
    return pl.pallas_call(
        _edge_body,
        grid=grid,
        in_specs=in_specs,
        out_specs=[pl.BlockSpec((2, GB // 2, 4 * 2 * C), lambda i: (0, i, 0))],
        out_shape=[jax.ShapeDtypeStruct((2, EPAD // 4, 4 * 2 * C),
                                        jnp.float32)],
    )(sndp, rcvf, rcvf, ev, ev, sef, sef, ln, ln,
      wm0p, bm0, wm1, bm1, wm2, bm2)[0]


def _scatter_body(m_hbm, rcv_hbm, z_hbm, agg_hbm, acc_sh, idx_v, data_v):
    c = lax.axis_index("c")
    s = lax.axis_index("s")
    stripe = s * STRIPE
    pltpu.sync_copy(z_hbm, acc_sh.at[pl.ds(stripe, STRIPE)])
    plsc.subcore_barrier()

    def body(j, carry):
        grow = s * ROWS_CORE + j
        pltpu.sync_copy(rcv_hbm.at[grow], idx_v)
        pltpu.sync_copy(m_hbm.at[c, pl.ds(grow * K, K)], data_v)
        pltpu.sync_copy(data_v, acc_sh.at[idx_v], add=True)
        return carry

    lax.fori_loop(0, ROWS_CORE, body, 0)
    plsc.subcore_barrier()
    pltpu.sync_copy(acc_sh.at[pl.ds(stripe, STRIPE)],
                    agg_hbm.at[c, pl.ds(stripe, STRIPE)])


@functools.cache
def _scatter():
    return pl.kernel(
        _scatter_body,
        out_type=jax.ShapeDtypeStruct((2, N, 2 * C), jnp.float32),
        mesh=_sc_mesh(),
        scratch_types=[
            pltpu.VMEM_SHARED((N, 2 * C), jnp.float32),
            pltpu.VMEM((K,), jnp.int32),
            pltpu.VMEM((K, 2 * C), jnp.float32),
        ],
        compiler_params=pltpu.CompilerParams(use_tc_tiling_on_sc=False),
    )


def _linear_dn_body(agg_ref, w0_ref, b0_ref, w1_ref, out_ref):
    a = agg_ref[...]
    a0 = a[0, :, :C]
    ax = a[0, :, C:]
    ay = a[1, :, :C]
    az = a[1, :, C:]
    scale = INV / AVG
    o0 = jnp.dot(a0, w0_ref[...],
                 preferred_element_type=jnp.float32) * scale + b0_ref[...]
    ox = jnp.dot(ax, w1_ref[...], preferred_element_type=jnp.float32) * scale
    oy = jnp.dot(ay, w1_ref[...], preferred_element_type=jnp.float32) * scale
    oz = jnp.dot(az, w1_ref[...], preferred_element_type=jnp.float32) * scale
    out_ref[...] = jnp.concatenate([o0, ox, oy, oz], axis=1)


def _linear_dn(agg, w0, b0, w1):
    grid = (N // BN,)
    return pl.pallas_call(
        _linear_dn_body,
        grid=grid,
        in_specs=[
            pl.BlockSpec((2, BN, 2 * C), lambda i: (0, i, 0)),
            pl.BlockSpec((C, C), lambda i: (0, 0)),
            pl.BlockSpec((1, C), lambda i: (0, 0)),
            pl.BlockSpec((C, C), lambda i: (0, 0)),
        ],
        out_specs=[pl.BlockSpec((BN, 4 * C), lambda i: (i, 0))],
        out_shape=[jax.ShapeDtypeStruct((N, 4 * C), jnp.float32)],
    )(agg, w0, b0, w1)[0]


def kernel(node_scalars, node_vectors, edge_vec, scalar_edge_feats, lengths,
           W0_up, b0_up, W1_up, Wm0, bm0, Wm1, bm1, Wm2, bm2,
           W0_dn, b0_dn, W1_dn, senders, receivers):
    nvx = node_vectors[:, :, 0]
    nvy = node_vectors[:, :, 1]
    nvz = node_vectors[:, :, 2]
    sv, s16 = _linear_up(node_scalars, nvx, nvy, nvz,
                         W0_up, b0_up.reshape(1, C), W1_up)

    padn = EPAD - E
    snd_pad = jnp.pad(senders, (0, padn))
    snd_perm = (snd_pad.reshape(NG, 2, GB).transpose(0, 2, 1)
                .reshape(NROWS, K))
    rcv_pad = jnp.pad(receivers, (0, padn))
    rcv2d = rcv_pad.reshape(NROWS, K)
    evp = jnp.pad(edge_vec, ((0, padn), (0, 0)))
    sefp = jnp.pad(scalar_edge_feats, ((0, padn), (0, 0)))
    lnp = jnp.pad(lengths, ((0, padn), (0, 0)))

    sndf, rcvf = _gather()(sv, s16, snd_perm, rcv2d)
    sndp = sndf.reshape(EPAD // 2, 2 * 4 * C)

    wm0p = jnp.concatenate([Wm0, jnp.zeros((3 * C - (2 * C + SE + 1), HID),
                                           jnp.float32)], axis=0)
    m4 = _edge_mlp(sndp, rcvf, evp, sefp, lnp,
                   wm0p, bm0.reshape(1, HID), Wm1, bm1.reshape(1, HID),
                   Wm2, bm2.reshape(1, 2 * C))
    m2 = m4.reshape(2, EPAD, 2 * C)
    rcv_s = (rcv_pad.reshape(NG, 4, GB // 2)[:, jnp.array([0, 2, 1, 3]), :]
             .transpose(0, 2, 1).reshape(NROWS, K))

    agg = _scatter()(m2, rcv_s, jnp.zeros((STRIPE, 2 * C), jnp.float32))

    outax = _linear_dn(agg, W0_dn, b0_dn.reshape(1, C), W1_dn)
    o1 = outax[:, C:].reshape(N, 3, C).transpose(0, 2, 1).reshape(N, 3 * C)
    return jnp.concatenate([outax[:, :C], o1], axis=1)


# speedup vs baseline: 24.0067x; 1.5655x over previous
import functools

import jax
import jax.numpy as jnp
import numpy as np
from jax import lax
from jax.experimental import pallas as pl
from jax.experimental.pallas import tpu as pltpu
from jax.experimental.pallas import tpu_sc as plsc

N = 50000
E = 800000
C = 16
SE = 8
HID = 128
AVG = 16.0
INV = 1.0 / np.sqrt(C)
ISQ3 = 1.0 / np.sqrt(3.0)

NC = 2
NS = 16
NW = NC * NS

K = 128
NROWS = 6272
EPAD = NROWS * K
ROWS_W = NROWS // NW
ROWS_CORE = NROWS // NS
STRIPE = N // NS

BN = 2000
GB = 1024
NG = EPAD // (2 * GB)


@functools.cache
def _sc_mesh():
    return plsc.VectorSubcoreMesh(
        core_axis_name="c", subcore_axis_name="s",
        num_cores=NC, num_subcores=NS)


def _linear_up_body(ns_ref, nvx_ref, nvy_ref, nvz_ref, w0_ref, b0_ref, w1_ref,
                    sv_ref, s_ref):
    s = jnp.dot(ns_ref[...], w0_ref[...],
                preferred_element_type=jnp.float32) * INV + b0_ref[...]
    vx = jnp.dot(nvx_ref[...], w1_ref[...],
                 preferred_element_type=jnp.float32) * INV
    vy = jnp.dot(nvy_ref[...], w1_ref[...],
                 preferred_element_type=jnp.float32) * INV
    vz = jnp.dot(nvz_ref[...], w1_ref[...],
                 preferred_element_type=jnp.float32) * INV
    sv_ref[...] = jnp.concatenate([s, vx, vy, vz], axis=1)
    s_ref[...] = s


def _linear_up(ns, nvx, nvy, nvz, w0, b0, w1):
    grid = (N // BN,)
    return pl.pallas_call(
        _linear_up_body,
        grid=grid,
        in_specs=[
            pl.BlockSpec((BN, C), lambda i: (i, 0)),
            pl.BlockSpec((BN, C), lambda i: (i, 0)),
            pl.BlockSpec((BN, C), lambda i: (i, 0)),
            pl.BlockSpec((BN, C), lambda i: (i, 0)),
            pl.BlockSpec((C, C), lambda i: (0, 0)),
            pl.BlockSpec((1, C), lambda i: (0, 0)),
            pl.BlockSpec((C, C), lambda i: (0, 0)),
        ],
        out_specs=[
            pl.BlockSpec((BN, 4 * C), lambda i: (i, 0)),
            pl.BlockSpec((BN, C), lambda i: (i, 0)),
        ],
        out_shape=[
            jax.ShapeDtypeStruct((N, 4 * C), jnp.float32),
            jax.ShapeDtypeStruct((N, C), jnp.float32),
        ],
    )(ns, nvx, nvy, nvz, w0, b0, w1)


def _gather_body(sv_hbm, s_hbm, snd_hbm, rcv_hbm, sndf_hbm, rcvf_hbm,
                 idx_v, rows64_v, rows16_v, sem):
    c = lax.axis_index("c")
    s = lax.axis_index("s")
    wid = s * NC + c
    base = wid * ROWS_W

    def body(j, carry):
        row = base + j
        pltpu.sync_copy(snd_hbm.at[row], idx_v)
        pltpu.async_copy(sv_hbm.at[idx_v], rows64_v, sem).wait()
        pltpu.sync_copy(rows64_v, sndf_hbm.at[pl.ds(row * K, K)])
        pltpu.sync_copy(rcv_hbm.at[row], idx_v)
        pltpu.async_copy(s_hbm.at[idx_v], rows16_v, sem).wait()
        pltpu.sync_copy(rows16_v, rcvf_hbm.at[pl.ds(row * K, K)])
        return carry

    lax.fori_loop(0, ROWS_W, body, 0)


@functools.cache
def _gather():
    return pl.kernel(
        _gather_body,
        out_type=[
            jax.ShapeDtypeStruct((EPAD, 4 * C), jnp.float32),
            jax.ShapeDtypeStruct((EPAD, C), jnp.float32),
        ],
        mesh=_sc_mesh(),
        scratch_types=[
            pltpu.VMEM((K,), jnp.int32),
            pltpu.VMEM((K, 4 * C), jnp.float32),
            pltpu.VMEM((K, C), jnp.float32),
            pltpu.SemaphoreType.DMA,
        ],
        compiler_params=pltpu.CompilerParams(use_tc_tiling_on_sc=False),
    )


def _edge_group(snd, rcv, ev, sef, ln,
                wm0_ref, bm0_ref, wm1_ref, bm1_ref, wm2_ref, bm2_ref):
    s_snd = snd[:, :C]
    vx = snd[:, C:2 * C]
    vy = snd[:, 2 * C:3 * C]
    vz = snd[:, 3 * C:4 * C]
    evx = ev[:, 0:1]
    evy = ev[:, 1:2]
    evz = ev[:, 2:3]

    pad = jnp.zeros((GB, 7), jnp.float32)
    mlp_in = jnp.concatenate([s_snd, rcv, sef, ln, pad], axis=1)

    h = jnp.dot(mlp_in, wm0_ref[...],
                preferred_element_type=jnp.float32) + bm0_ref[...]
    h = h * (1.0 / (1.0 + jnp.exp(-h)))
    h = jnp.dot(h, wm1_ref[...],
                preferred_element_type=jnp.float32) + bm1_ref[...]
    h = h * (1.0 / (1.0 + jnp.exp(-h)))
    mix = jnp.dot(h, wm2_ref[...],
                  preferred_element_type=jnp.float32) + bm2_ref[...]

    mix_lo = mix[:, :C]
    mix_hi = mix[:, C:]
    m0 = (vx * evx + vy * evy + vz * evz) * (mix_lo * ISQ3)
    sm = s_snd * mix_hi
    return (jnp.concatenate([m0, sm * evx], axis=1),
            jnp.concatenate([sm * evy, sm * evz], axis=1))


def _edge_body(snd_ref, rcv0_ref, rcv1_ref, ev0_ref, ev1_ref,
               sef0_ref, sef1_ref, ln0_ref, ln1_ref,
               wm0_ref, bm0_ref, wm1_ref, bm1_ref, wm2_ref, bm2_ref, m_ref):
    snd = snd_ref[...]
    wrefs = (wm0_ref, bm0_ref, wm1_ref, bm1_ref, wm2_ref, bm2_ref)
    h0e, h1e = _edge_group(snd[:, :4 * C], rcv0_ref[...], ev0_ref[...].T,
                           sef0_ref[...].T, ln0_ref[...].T, *wrefs)
    h0o, h1o = _edge_group(snd[:, 4 * C:], rcv1_ref[...], ev1_ref[...].T,
                           sef1_ref[...].T, ln1_ref[...].T, *wrefs)
    hb = GB // 2
    m_ref[0] = jnp.concatenate(
        [h0e[:hb], h0o[:hb], h0e[hb:], h0o[hb:]], axis=1)
    m_ref[1] = jnp.concatenate(
        [h1e[:hb], h1o[:hb], h1e[hb:], h1o[hb:]], axis=1)


def _edge_mlp(sndp, rcvf, ev, sef, ln, wm0p, bm0, wm1, bm1, wm2, bm2):
    grid = (NG,)
    wspec = lambda shape: pl.BlockSpec(shape, lambda i: (0,) * len(shape))

    def gspec(width, g):
        return pl.BlockSpec((GB, width), lambda i, g=g: (2 * i + g, 0))

    def tspec(height, g):
        return pl.BlockSpec((height, GB), lambda i, g=g: (0, 2 * i + g))

    in_specs = (
        [pl.BlockSpec((GB, 2 * 4 * C), lambda i: (i, 0))]
        + [gspec(C, g) for g in range(2)]
        + [tspec(3, g) for g in range(2)]
        + [tspec(SE, g) for g in range(2)]
        + [tspec(1, g) for g in range(2)]
        + [wspec((3 * C, HID)), wspec((1, HID)), wspec((HID, HID)),
           wspec((1, HID)), wspec((HID, 2 * C)), wspec((1, 2 * C))]
    )
    return pl.pallas_call(
        _edge_body,
        grid=grid,
        in_specs=in_specs,
        out_specs=[pl.BlockSpec((2, GB // 2, 4 * 2 * C), lambda i: (0, i, 0))],
        out_shape=[jax.ShapeDtypeStruct((2, EPAD // 4, 4 * 2 * C),
                                        jnp.float32)],
    )(sndp, rcvf, rcvf, ev, ev, sef, sef, ln, ln,
      wm0p, bm0, wm1, bm1, wm2, bm2)[0]


def _scatter_body(m_hbm, rcv_hbm, z_hbm, agg_hbm, acc_sh, idx_v, data_v):
    c = lax.axis_index("c")
    s = lax.axis_index("s")
    stripe = s * STRIPE
    pltpu.sync_copy(z_hbm, acc_sh.at[pl.ds(stripe, STRIPE)])
    plsc.subcore_barrier()

    def body(j, carry):
        grow = s * ROWS_CORE + j
        pltpu.sync_copy(rcv_hbm.at[grow], idx_v)
        pltpu.sync_copy(m_hbm.at[c, pl.ds(grow * K, K)], data_v)
        pltpu.sync_copy(data_v, acc_sh.at[idx_v], add=True)
        return carry

    lax.fori_loop(0, ROWS_CORE, body, 0)
    plsc.subcore_barrier()
    pltpu.sync_copy(acc_sh.at[pl.ds(stripe, STRIPE)],
                    agg_hbm.at[c, pl.ds(stripe, STRIPE)])


@functools.cache
def _scatter():
    return pl.kernel(
        _scatter_body,
        out_type=jax.ShapeDtypeStruct((2, N, 2 * C), jnp.float32),
        mesh=_sc_mesh(),
        scratch_types=[
            pltpu.VMEM_SHARED((N, 2 * C), jnp.float32),
            pltpu.VMEM((K,), jnp.int32),
            pltpu.VMEM((K, 2 * C), jnp.float32),
        ],
        compiler_params=pltpu.CompilerParams(use_tc_tiling_on_sc=False),
    )


def _linear_dn_body(agg_ref, w0_ref, b0_ref, w1_ref, out_ref):
    a = agg_ref[...]
    a0 = a[0, :, :C]
    ax = a[0, :, C:]
    ay = a[1, :, :C]
    az = a[1, :, C:]
    scale = INV / AVG
    o0 = jnp.dot(a0, w0_ref[...],
                 preferred_element_type=jnp.float32) * scale + b0_ref[...]
    ox = jnp.dot(ax, w1_ref[...], preferred_element_type=jnp.float32) * scale
    oy = jnp.dot(ay, w1_ref[...], preferred_element_type=jnp.float32) * scale
    oz = jnp.dot(az, w1_ref[...], preferred_element_type=jnp.float32) * scale
    out_ref[...] = jnp.concatenate([o0, ox, oy, oz], axis=1)


def _linear_dn(agg, w0, b0, w1):
    grid = (N // BN,)
    return pl.pallas_call(
        _linear_dn_body,
        grid=grid,
        in_specs=[
            pl.BlockSpec((2, BN, 2 * C), lambda i: (0, i, 0)),
            pl.BlockSpec((C, C), lambda i: (0, 0)),
            pl.BlockSpec((1, C), lambda i: (0, 0)),
            pl.BlockSpec((C, C), lambda i: (0, 0)),
        ],
        out_specs=[pl.BlockSpec((BN, 4 * C), lambda i: (i, 0))],
        out_shape=[jax.ShapeDtypeStruct((N, 4 * C), jnp.float32)],
    )(agg, w0, b0, w1)[0]


def kernel(node_scalars, node_vectors, edge_vec, scalar_edge_feats, lengths,
           W0_up, b0_up, W1_up, Wm0, bm0, Wm1, bm1, Wm2, bm2,
           W0_dn, b0_dn, W1_dn, senders, receivers):
    nvx = node_vectors[:, :, 0]
    nvy = node_vectors[:, :, 1]
    nvz = node_vectors[:, :, 2]
    sv, s16 = _linear_up(node_scalars, nvx, nvy, nvz,
                         W0_up, b0_up.reshape(1, C), W1_up)

    padn = EPAD - E
    snd_pad = jnp.pad(senders, (0, padn))
    snd_perm = (snd_pad.reshape(NG, 2, GB).transpose(0, 2, 1)
                .reshape(NROWS, K))
    rcv_pad = jnp.pad(receivers, (0, padn))
    rcv2d = rcv_pad.reshape(NROWS, K)
    evp = jnp.pad(edge_vec.T, ((0, 0), (0, padn)))
    sefp = jnp.pad(scalar_edge_feats.T, ((0, 0), (0, padn)))
    lnp = jnp.pad(lengths.T, ((0, 0), (0, padn)))

    sndf, rcvf = _gather()(sv, s16, snd_perm, rcv2d)
    sndp = sndf.reshape(EPAD // 2, 2 * 4 * C)

    wm0p = jnp.concatenate([Wm0, jnp.zeros((3 * C - (2 * C + SE + 1), HID),
                                           jnp.float32)], axis=0)
    m4 = _edge_mlp(sndp, rcvf, evp, sefp, lnp,
                   wm0p, bm0.reshape(1, HID), Wm1, bm1.reshape(1, HID),
                   Wm2, bm2.reshape(1, 2 * C))
    m2 = m4.reshape(2, EPAD, 2 * C)
    rcv_s = (rcv_pad.reshape(NG, 4, GB // 2)[:, jnp.array([0, 2, 1, 3]), :]
             .transpose(0, 2, 1).reshape(NROWS, K))

    agg = _scatter()(m2, rcv_s, jnp.zeros((STRIPE, 2 * C), jnp.float32))

    outax = _linear_dn(agg, W0_dn, b0_dn.reshape(1, C), W1_dn)
    o1 = outax[:, C:].reshape(N, 3, C).transpose(0, 2, 1).reshape(N, 3 * C)
    return jnp.concatenate([outax[:, :C], o1], axis=1)
